# baseline (speedup 1.0000x reference)
import functools

import jax
import jax.numpy as jnp
from jax import lax
from jax.experimental import pallas as pl
from jax.experimental.pallas import tpu as pltpu
from jax.experimental.pallas import tpu_sc as plsc

N = 10000
E = 320000
FD = 128
OUT = 64
K1 = 16
K2 = 8

NCORES = 2
NSUB = 16
NW = NCORES * NSUB
CHUNK = 128
NCH = 80
TPW = NCH * CHUNK
EPAD = TPW * NW
NACC = 10240
RPT = NACC // NSUB

RB = 1000
GRID = N // RB

def _zero_shared(zeros_hbm, acc_sh, s):
    base = s * RPT
    nfull = RPT // CHUNK
    for m in range(nfull):
        pltpu.sync_copy(zeros_hbm, acc_sh.at[pl.ds(base + m * CHUNK, CHUNK)])
    rem = RPT - nfull * CHUNK
    if rem:
        pltpu.sync_copy(zeros_hbm.at[pl.ds(0, rem)],
                        acc_sh.at[pl.ds(base + nfull * CHUNK, rem)])


@functools.cache
def _sc_kernels():
    mesh = plsc.VectorSubcoreMesh(core_axis_name="c", subcore_axis_name="s",
                                  num_cores=NCORES, num_subcores=NSUB)

    @functools.partial(
        pl.kernel,
        out_type=jax.ShapeDtypeStruct((NCORES, NACC, FD), jnp.float32),
        mesh=mesh,
        scratch_types=[
            pltpu.VMEM((NCH, CHUNK), jnp.int32),
            pltpu.VMEM((CHUNK,), jnp.int32),
            pltpu.VMEM((CHUNK,), jnp.int32),
            pltpu.VMEM((CHUNK, FD), jnp.float32),
            pltpu.VMEM((CHUNK, FD), jnp.float32),
            pltpu.VMEM_SHARED((NACC, FD), jnp.float32),
            pltpu.SemaphoreType.DMA,
            pltpu.SemaphoreType.DMA,
        ],
    )
    def spmm_sc(g_hbm, sidx_hbm, didx_hbm, zeros_hbm, out_hbm,
                sidx_v, db_a, db_b, rows_a, rows_b, acc_sh, gsem_a, gsem_b):
        c = lax.axis_index("c")
        s = lax.axis_index("s")
        wid = s * NCORES + c
        _zero_shared(zeros_hbm, acc_sh, s)
        pltpu.sync_copy(sidx_hbm.at[wid], sidx_v)
        plsc.subcore_barrier()

        pltpu.sync_copy(didx_hbm.at[wid, 0], db_a)
        pltpu.async_copy(g_hbm.at[sidx_v.at[0]], rows_a, gsem_a)
        pltpu.sync_copy(didx_hbm.at[wid, 1], db_b)

        def body(t, carry):
            j0 = 2 * t
            j2 = jnp.minimum(j0 + 2, NCH - 1)
            j3 = jnp.minimum(j0 + 3, NCH - 1)
            pltpu.make_async_copy(zeros_hbm, rows_a, gsem_a).wait()
            pltpu.async_copy(g_hbm.at[sidx_v.at[j0 + 1]], rows_b, gsem_b)
            pltpu.sync_copy(rows_a, acc_sh.at[db_a], add=True)
            pltpu.sync_copy(didx_hbm.at[wid, j2], db_a)
            pltpu.async_copy(g_hbm.at[sidx_v.at[j2]], rows_a, gsem_a)
            pltpu.make_async_copy(zeros_hbm, rows_b, gsem_b).wait()
            pltpu.sync_copy(rows_b, acc_sh.at[db_b], add=True)
            pltpu.sync_copy(didx_hbm.at[wid, j3], db_b)
            return carry

        lax.fori_loop(0, NCH // 2, body, 0)
        pltpu.make_async_copy(zeros_hbm, rows_a, gsem_a).wait()
        plsc.subcore_barrier()
        pltpu.sync_copy(acc_sh.at[pl.ds(s * RPT, RPT)],
                        out_hbm.at[c, pl.ds(s * RPT, RPT)])

    return spmm_sc


def _row_spec(width=FD):
    return pl.BlockSpec((RB, width), lambda i: (i, 0))


def _prep(degp, x):
    def body(d0_ref, d1_ref, x_ref, dis_ref, g_ref):
        deg = d0_ref[0][:, 0:1] + d1_ref[0][:, 0:1]
        dis = jnp.where(deg > 0.0, lax.rsqrt(jnp.maximum(deg, 1e-12)), 0.0)
        dis_ref[...] = dis
        g_ref[...] = dis * x_ref[...]

    return pl.pallas_call(
        body,
        grid=(GRID,),
        in_specs=[
            pl.BlockSpec((1, RB, FD), lambda i: (0, i, 0)),
            pl.BlockSpec((1, RB, FD), lambda i: (1, i, 0)),
            _row_spec(),
        ],
        out_specs=[pl.BlockSpec((RB, 1), lambda i: (i, 0)), _row_spec()],
        out_shape=[
            jax.ShapeDtypeStruct((N, 1), jnp.float32),
            jax.ShapeDtypeStruct((N, FD), jnp.float32),
        ],
    )(degp, degp, x)


def _p_specs():
    return [
        pl.BlockSpec((1, RB, FD), lambda i: (0, i, 0)),
        pl.BlockSpec((1, RB, FD), lambda i: (1, i, 0)),
    ]


def _combine_first(p, dis):
    def body(p0_ref, p1_ref, dis_ref, tx_ref, g_ref):
        t = -(dis_ref[...] * (p0_ref[0] + p1_ref[0]))
        tx_ref[...] = t
        g_ref[...] = dis_ref[...] * t

    return pl.pallas_call(
        body,
        grid=(GRID,),
        in_specs=_p_specs() + [pl.BlockSpec((RB, 1), lambda i: (i, 0))],
        out_specs=[_row_spec(), _row_spec()],
        out_shape=[
            jax.ShapeDtypeStruct((N, FD), jnp.float32),
            jax.ShapeDtypeStruct((N, FD), jnp.float32),
        ],
    )(p, p, dis)


def _combine_rec(p, dis, tx0):
    def body(p0_ref, p1_ref, dis_ref, tx0_ref, tx_ref, g_ref):
        t = -2.0 * (dis_ref[...] * (p0_ref[0] + p1_ref[0])) - tx0_ref[...]
        tx_ref[...] = t
        g_ref[...] = dis_ref[...] * t

    return pl.pallas_call(
        body,
        grid=(GRID,),
        in_specs=_p_specs() + [pl.BlockSpec((RB, 1), lambda i: (i, 0)),
                               _row_spec()],
        out_specs=[_row_spec(), _row_spec()],
        out_shape=[
            jax.ShapeDtypeStruct((N, FD), jnp.float32),
            jax.ShapeDtypeStruct((N, FD), jnp.float32),
        ],
    )(p, p, dis, tx0)


def _mm_stats(basis, W):
    K = len(basis)

    def body(*refs):
        tx_refs = refs[:K]
        w_ref = refs[K]
        h_ref, st_ref = refs[K + 1], refs[K + 2]
        acc_ref = refs[K + 3]
        i = pl.program_id(0)
        h = jnp.zeros((RB, FD), jnp.float32)
        for k in range(K):
            h = h + jnp.dot(tx_refs[k][...], w_ref[k],
                            preferred_element_type=jnp.float32)
        h_ref[...] = h

        @pl.when(i == 0)
        def _():
            acc_ref[...] = jnp.zeros((8, FD), jnp.float32)

        acc_ref[0:1, :] += jnp.sum(h, axis=0, keepdims=True)
        acc_ref[1:2, :] += jnp.sum(h * h, axis=0, keepdims=True)
        st_ref[...] = acc_ref[...]

    return pl.pallas_call(
        body,
        grid=(GRID,),
        in_specs=[_row_spec() for _ in range(K)]
        + [pl.BlockSpec((K, FD, FD), lambda i: (0, 0, 0))],
        out_specs=[_row_spec(), pl.BlockSpec((8, FD), lambda i: (0, 0))],
        out_shape=[
            jax.ShapeDtypeStruct((N, FD), jnp.float32),
            jax.ShapeDtypeStruct((8, FD), jnp.float32),
        ],
        scratch_shapes=[pltpu.VMEM((8, FD), jnp.float32)],
    )(*basis, W)


def _bn_relu_scale(h, st, dis):
    def body(h_ref, st_ref, dis_ref, out_ref, g_ref):
        mean = st_ref[0:1, :] * (1.0 / N)
        var = st_ref[1:2, :] * (1.0 / N) - mean * mean
        inv = lax.rsqrt(var + 1e-5)
        hr = jnp.maximum((h_ref[...] - mean) * inv, 0.0)
        out_ref[...] = hr
        g_ref[...] = dis_ref[...] * hr

    return pl.pallas_call(
        body,
        grid=(GRID,),
        in_specs=[_row_spec(), pl.BlockSpec((8, FD), lambda i: (0, 0)),
                  pl.BlockSpec((RB, 1), lambda i: (i, 0))],
        out_specs=[_row_spec(), _row_spec()],
        out_shape=[
            jax.ShapeDtypeStruct((N, FD), jnp.float32),
            jax.ShapeDtypeStruct((N, FD), jnp.float32),
        ],
    )(h, st, dis)


def _bn(h, st):
    def body(h_ref, st_ref, out_ref):
        mean = st_ref[0:1, :] * (1.0 / N)
        var = st_ref[1:2, :] * (1.0 / N) - mean * mean
        inv = lax.rsqrt(var + 1e-5)
        out_ref[...] = (h_ref[...] - mean) * inv

    return pl.pallas_call(
        body,
        grid=(GRID,),
        in_specs=[_row_spec(), pl.BlockSpec((8, FD), lambda i: (0, 0))],
        out_specs=_row_spec(),
        out_shape=jax.ShapeDtypeStruct((N, FD), jnp.float32),
    )(h, st)


def kernel(x, edge_index, W1, b1, Wmu, bmu, Wls, bls):
    src = edge_index[0]
    dst = edge_index[1]
    pad = EPAD - E
    pad_zero = jnp.zeros((pad,), jnp.int32)
    pad_dummy = jnp.full((pad,), N, jnp.int32)
    sidx = jnp.concatenate([src, pad_zero]).reshape(NW, NCH, CHUNK)
    didx = jnp.concatenate([dst, pad_dummy]).reshape(NW, NCH, CHUNK)
    sdeg = jnp.concatenate([src, pad_dummy]).reshape(NW, NCH, CHUNK)
    zeros128 = jnp.zeros((CHUNK, FD), jnp.float32)
    _spmm_sc = _sc_kernels()

    ones_tab = jnp.ones((8, FD), jnp.float32)
    zidx = jnp.zeros((NW, NCH, CHUNK), jnp.int32)
    degp = _spmm_sc(ones_tab, zidx, sdeg, zeros128)
    dis, g = _prep(degp, x)

    basis = [x]
    p = _spmm_sc(g, sidx, didx, zeros128)
    tx, g = _combine_first(p, dis)
    basis.append(tx)
    txm2, txm1 = basis[0], tx
    for _ in range(2, K1):
        p = _spmm_sc(g, sidx, didx, zeros128)
        tx, g = _combine_rec(p, dis, txm2)
        basis.append(tx)
        txm2, txm1 = txm1, tx
    h, st = _mm_stats(basis, W1)
    hr, g = _bn_relu_scale(h, st, dis)

    basis2 = [hr]
    p = _spmm_sc(g, sidx, didx, zeros128)
    tx, g = _combine_first(p, dis)
    basis2.append(tx)
    txm2, txm1 = basis2[0], tx
    for _ in range(2, K2):
        p = _spmm_sc(g, sidx, didx, zeros128)
        tx, g = _combine_rec(p, dis, txm2)
        basis2.append(tx)
        txm2, txm1 = txm1, tx
    Wc = jnp.concatenate([Wmu, Wls], axis=-1)
    h2, st2 = _mm_stats(basis2, Wc)
    out = _bn(h2, st2)
    return out[:, :OUT], out[:, OUT:]


# speedup vs baseline: 2.1138x; 2.1138x over previous
import functools

import jax
import jax.numpy as jnp
from jax import lax
from jax.experimental import pallas as pl
from jax.experimental.pallas import tpu as pltpu
from jax.experimental.pallas import tpu_sc as plsc

N = 10000
E = 320000
FD = 128
OUT = 64
K1 = 16
K2 = 8

NCORES = 2
NSUB = 16
NW = NCORES * NSUB
CHUNK = 128
NCH = 80
TPW = NCH * CHUNK
EPAD = TPW * NW
NACC = 10240
RPT = NACC // NSUB

RB = 1000
GRID = N // RB

def _zero_shared(zeros_hbm, acc_sh, s):
    base = s * RPT
    nfull = RPT // CHUNK
    for m in range(nfull):
        pltpu.sync_copy(zeros_hbm, acc_sh.at[pl.ds(base + m * CHUNK, CHUNK)])
    rem = RPT - nfull * CHUNK
    if rem:
        pltpu.sync_copy(zeros_hbm.at[pl.ds(0, rem)],
                        acc_sh.at[pl.ds(base + nfull * CHUNK, rem)])


@functools.cache
def _sc_kernels():
    mesh = plsc.VectorSubcoreMesh(core_axis_name="c", subcore_axis_name="s",
                                  num_cores=NCORES, num_subcores=NSUB)

    @functools.partial(
        pl.kernel,
        out_type=jax.ShapeDtypeStruct((NCORES, NACC, FD), jnp.float32),
        mesh=mesh,
        scratch_types=[
            pltpu.VMEM((NCH, CHUNK), jnp.int32),
            pltpu.VMEM((NCH, CHUNK), jnp.int32),
            pltpu.VMEM((CHUNK, FD), jnp.float32),
            pltpu.VMEM_SHARED((NACC, FD), jnp.float32),
            pltpu.SemaphoreType.DMA,
        ],
    )
    def spmm_sc(g_hbm, sidx_hbm, didx_hbm, zeros_hbm, out_hbm,
                sidx_v, didx_v, rows_v, acc_sh, sem):
        c = lax.axis_index("c")
        s = lax.axis_index("s")
        wid = s * NCORES + c
        _zero_shared(zeros_hbm, acc_sh, s)
        pltpu.sync_copy(sidx_hbm.at[wid], sidx_v)
        pltpu.sync_copy(didx_hbm.at[wid], didx_v)
        plsc.subcore_barrier()

        def body(j, carry):
            pltpu.async_copy(g_hbm.at[sidx_v.at[j]], rows_v, sem).wait()
            pltpu.sync_copy(rows_v, acc_sh.at[didx_v.at[j]], add=True)
            return carry

        lax.fori_loop(0, NCH, body, 0)
        plsc.subcore_barrier()
        pltpu.sync_copy(acc_sh.at[pl.ds(s * RPT, RPT)],
                        out_hbm.at[c, pl.ds(s * RPT, RPT)])

    return spmm_sc


def _row_spec(width=FD):
    return pl.BlockSpec((RB, width), lambda i: (i, 0))


def _prep(degp, x):
    def body(d0_ref, d1_ref, x_ref, dis_ref, g_ref):
        deg = d0_ref[0][:, 0:1] + d1_ref[0][:, 0:1]
        dis = jnp.where(deg > 0.0, lax.rsqrt(jnp.maximum(deg, 1e-12)), 0.0)
        dis_ref[...] = dis
        g_ref[...] = dis * x_ref[...]

    return pl.pallas_call(
        body,
        grid=(GRID,),
        in_specs=[
            pl.BlockSpec((1, RB, FD), lambda i: (0, i, 0)),
            pl.BlockSpec((1, RB, FD), lambda i: (1, i, 0)),
            _row_spec(),
        ],
        out_specs=[pl.BlockSpec((RB, 1), lambda i: (i, 0)), _row_spec()],
        out_shape=[
            jax.ShapeDtypeStruct((N, 1), jnp.float32),
            jax.ShapeDtypeStruct((N, FD), jnp.float32),
        ],
    )(degp, degp, x)


def _p_specs():
    return [
        pl.BlockSpec((1, RB, FD), lambda i: (0, i, 0)),
        pl.BlockSpec((1, RB, FD), lambda i: (1, i, 0)),
    ]


def _combine_first(p, dis):
    def body(p0_ref, p1_ref, dis_ref, tx_ref, g_ref):
        t = -(dis_ref[...] * (p0_ref[0] + p1_ref[0]))
        tx_ref[...] = t
        g_ref[...] = dis_ref[...] * t

    return pl.pallas_call(
        body,
        grid=(GRID,),
        in_specs=_p_specs() + [pl.BlockSpec((RB, 1), lambda i: (i, 0))],
        out_specs=[_row_spec(), _row_spec()],
        out_shape=[
            jax.ShapeDtypeStruct((N, FD), jnp.float32),
            jax.ShapeDtypeStruct((N, FD), jnp.float32),
        ],
    )(p, p, dis)


def _combine_rec(p, dis, tx0):
    def body(p0_ref, p1_ref, dis_ref, tx0_ref, tx_ref, g_ref):
        t = -2.0 * (dis_ref[...] * (p0_ref[0] + p1_ref[0])) - tx0_ref[...]
        tx_ref[...] = t
        g_ref[...] = dis_ref[...] * t

    return pl.pallas_call(
        body,
        grid=(GRID,),
        in_specs=_p_specs() + [pl.BlockSpec((RB, 1), lambda i: (i, 0)),
                               _row_spec()],
        out_specs=[_row_spec(), _row_spec()],
        out_shape=[
            jax.ShapeDtypeStruct((N, FD), jnp.float32),
            jax.ShapeDtypeStruct((N, FD), jnp.float32),
        ],
    )(p, p, dis, tx0)


def _mm_stats(basis, W):
    K = len(basis)

    def body(*refs):
        tx_refs = refs[:K]
        w_ref = refs[K]
        h_ref, st_ref = refs[K + 1], refs[K + 2]
        acc_ref = refs[K + 3]
        i = pl.program_id(0)
        h = jnp.zeros((RB, FD), jnp.float32)
        for k in range(K):
            h = h + jnp.dot(tx_refs[k][...], w_ref[k],
                            preferred_element_type=jnp.float32)
        h_ref[...] = h

        @pl.when(i == 0)
        def _():
            acc_ref[...] = jnp.zeros((8, FD), jnp.float32)

        acc_ref[0:1, :] += jnp.sum(h, axis=0, keepdims=True)
        acc_ref[1:2, :] += jnp.sum(h * h, axis=0, keepdims=True)
        st_ref[...] = acc_ref[...]

    return pl.pallas_call(
        body,
        grid=(GRID,),
        in_specs=[_row_spec() for _ in range(K)]
        + [pl.BlockSpec((K, FD, FD), lambda i: (0, 0, 0))],
        out_specs=[_row_spec(), pl.BlockSpec((8, FD), lambda i: (0, 0))],
        out_shape=[
            jax.ShapeDtypeStruct((N, FD), jnp.float32),
            jax.ShapeDtypeStruct((8, FD), jnp.float32),
        ],
        scratch_shapes=[pltpu.VMEM((8, FD), jnp.float32)],
    )(*basis, W)


def _bn_relu_scale(h, st, dis):
    def body(h_ref, st_ref, dis_ref, out_ref, g_ref):
        mean = st_ref[0:1, :] * (1.0 / N)
        var = st_ref[1:2, :] * (1.0 / N) - mean * mean
        inv = lax.rsqrt(var + 1e-5)
        hr = jnp.maximum((h_ref[...] - mean) * inv, 0.0)
        out_ref[...] = hr
        g_ref[...] = dis_ref[...] * hr

    return pl.pallas_call(
        body,
        grid=(GRID,),
        in_specs=[_row_spec(), pl.BlockSpec((8, FD), lambda i: (0, 0)),
                  pl.BlockSpec((RB, 1), lambda i: (i, 0))],
        out_specs=[_row_spec(), _row_spec()],
        out_shape=[
            jax.ShapeDtypeStruct((N, FD), jnp.float32),
            jax.ShapeDtypeStruct((N, FD), jnp.float32),
        ],
    )(h, st, dis)


def _bn(h, st):
    def body(h_ref, st_ref, out_ref):
        mean = st_ref[0:1, :] * (1.0 / N)
        var = st_ref[1:2, :] * (1.0 / N) - mean * mean
        inv = lax.rsqrt(var + 1e-5)
        out_ref[...] = (h_ref[...] - mean) * inv

    return pl.pallas_call(
        body,
        grid=(GRID,),
        in_specs=[_row_spec(), pl.BlockSpec((8, FD), lambda i: (0, 0))],
        out_specs=_row_spec(),
        out_shape=jax.ShapeDtypeStruct((N, FD), jnp.float32),
    )(h, st)


def kernel(x, edge_index, W1, b1, Wmu, bmu, Wls, bls):
    src = edge_index[0]
    dst = edge_index[1]
    pad = EPAD - E
    pad_zero = jnp.zeros((pad,), jnp.int32)
    pad_dummy = jnp.full((pad,), N, jnp.int32)
    sidx = jnp.concatenate([src, pad_zero]).reshape(NW, NCH, CHUNK)
    didx = jnp.concatenate([dst, pad_dummy]).reshape(NW, NCH, CHUNK)
    sdeg = jnp.concatenate([src, pad_dummy]).reshape(NW, NCH, CHUNK)
    zeros128 = jnp.zeros((CHUNK, FD), jnp.float32)
    _spmm_sc = _sc_kernels()

    ones_tab = jnp.ones((N, FD), jnp.float32)
    degp = _spmm_sc(ones_tab, sidx, sdeg, zeros128)
    dis, g = _prep(degp, x)

    basis = [x]
    p = _spmm_sc(g, sidx, didx, zeros128)
    tx, g = _combine_first(p, dis)
    basis.append(tx)
    txm2, txm1 = basis[0], tx
    for _ in range(2, K1):
        p = _spmm_sc(g, sidx, didx, zeros128)
        tx, g = _combine_rec(p, dis, txm2)
        basis.append(tx)
        txm2, txm1 = txm1, tx
    h, st = _mm_stats(basis, W1)
    hr, g = _bn_relu_scale(h, st, dis)

    basis2 = [hr]
    p = _spmm_sc(g, sidx, didx, zeros128)
    tx, g = _combine_first(p, dis)
    basis2.append(tx)
    txm2, txm1 = basis2[0], tx
    for _ in range(2, K2):
        p = _spmm_sc(g, sidx, didx, zeros128)
        tx, g = _combine_rec(p, dis, txm2)
        basis2.append(tx)
        txm2, txm1 = txm1, tx
    Wc = jnp.concatenate([Wmu, Wls], axis=-1)
    h2, st2 = _mm_stats(basis2, Wc)
    out = _bn(h2, st2)
    return out[:, :OUT], out[:, OUT:]


# speedup vs baseline: 2.1738x; 1.0284x over previous
import functools

import jax
import jax.numpy as jnp
from jax import lax
from jax.experimental import pallas as pl
from jax.experimental.pallas import tpu as pltpu
from jax.experimental.pallas import tpu_sc as plsc

N = 10000
E = 320000
FD = 128
OUT = 64
K1 = 16
K2 = 8

NCORES = 2
NSUB = 16
NW = NCORES * NSUB
CHUNK = 128
NCH = 80
TPW = NCH * CHUNK
EPAD = TPW * NW
NACC = 10240
RPT = NACC // NSUB
GRP = 2

RB = 1000
GRID = N // RB

def _zero_shared(zeros_hbm, acc_sh, s):
    base = s * RPT
    nfull = RPT // CHUNK
    for m in range(nfull):
        pltpu.sync_copy(zeros_hbm, acc_sh.at[pl.ds(base + m * CHUNK, CHUNK)])
    rem = RPT - nfull * CHUNK
    if rem:
        pltpu.sync_copy(zeros_hbm.at[pl.ds(0, rem)],
                        acc_sh.at[pl.ds(base + nfull * CHUNK, rem)])


@functools.cache
def _sc_kernels():
    mesh = plsc.VectorSubcoreMesh(core_axis_name="c", subcore_axis_name="s",
                                  num_cores=NCORES, num_subcores=NSUB)

    @functools.partial(
        pl.kernel,
        out_type=jax.ShapeDtypeStruct((NCORES, NACC, FD), jnp.float32),
        mesh=mesh,
        scratch_types=[
            pltpu.VMEM((TPW,), jnp.int32),
            pltpu.VMEM((GRP * CHUNK,), jnp.int32),
            pltpu.VMEM((GRP * CHUNK, FD), jnp.float32),
            pltpu.VMEM_SHARED((NACC, FD), jnp.float32),
            pltpu.SemaphoreType.DMA,
            pltpu.SemaphoreType.DMA,
        ],
    )
    def spmm_sc(g_hbm, sidx_hbm, didx_hbm, zeros_hbm, out_hbm,
                sidx_v, didx_v, rows_v, acc_sh, gsem, isem):
        c = lax.axis_index("c")
        s = lax.axis_index("s")
        wid = s * NCORES + c
        _zero_shared(zeros_hbm, acc_sh, s)
        pltpu.sync_copy(sidx_hbm.at[wid], sidx_v)
        plsc.subcore_barrier()

        GC = GRP * CHUNK

        def body(t, carry):
            e0 = t * GC
            idma = pltpu.async_copy(didx_hbm.at[wid, pl.ds(e0, GC)],
                                    didx_v, isem)
            pltpu.async_copy(g_hbm.at[sidx_v.at[pl.ds(e0, GC)]],
                             rows_v, gsem).wait()
            idma.wait()
            pltpu.sync_copy(rows_v, acc_sh.at[didx_v], add=True)
            return carry

        lax.fori_loop(0, TPW // GC, body, 0)
        plsc.subcore_barrier()
        pltpu.sync_copy(acc_sh.at[pl.ds(s * RPT, RPT)],
                        out_hbm.at[c, pl.ds(s * RPT, RPT)])

    return spmm_sc


def _row_spec(width=FD):
    return pl.BlockSpec((RB, width), lambda i: (i, 0))


def _prep(degp, x):
    def body(d0_ref, d1_ref, x_ref, dis_ref, g_ref):
        deg = d0_ref[0][:, 0:1] + d1_ref[0][:, 0:1]
        dis = jnp.where(deg > 0.0, lax.rsqrt(jnp.maximum(deg, 1e-12)), 0.0)
        dis_ref[...] = dis
        g_ref[...] = dis * x_ref[...]

    return pl.pallas_call(
        body,
        grid=(GRID,),
        in_specs=[
            pl.BlockSpec((1, RB, FD), lambda i: (0, i, 0)),
            pl.BlockSpec((1, RB, FD), lambda i: (1, i, 0)),
            _row_spec(),
        ],
        out_specs=[pl.BlockSpec((RB, 1), lambda i: (i, 0)), _row_spec()],
        out_shape=[
            jax.ShapeDtypeStruct((N, 1), jnp.float32),
            jax.ShapeDtypeStruct((N, FD), jnp.float32),
        ],
    )(degp, degp, x)


def _p_specs():
    return [
        pl.BlockSpec((1, RB, FD), lambda i: (0, i, 0)),
        pl.BlockSpec((1, RB, FD), lambda i: (1, i, 0)),
    ]


def _combine_first(p, dis):
    def body(p0_ref, p1_ref, dis_ref, tx_ref, g_ref):
        t = -(dis_ref[...] * (p0_ref[0] + p1_ref[0]))
        tx_ref[...] = t
        g_ref[...] = dis_ref[...] * t

    return pl.pallas_call(
        body,
        grid=(GRID,),
        in_specs=_p_specs() + [pl.BlockSpec((RB, 1), lambda i: (i, 0))],
        out_specs=[_row_spec(), _row_spec()],
        out_shape=[
            jax.ShapeDtypeStruct((N, FD), jnp.float32),
            jax.ShapeDtypeStruct((N, FD), jnp.float32),
        ],
    )(p, p, dis)


def _combine_rec(p, dis, tx0):
    def body(p0_ref, p1_ref, dis_ref, tx0_ref, tx_ref, g_ref):
        t = -2.0 * (dis_ref[...] * (p0_ref[0] + p1_ref[0])) - tx0_ref[...]
        tx_ref[...] = t
        g_ref[...] = dis_ref[...] * t

    return pl.pallas_call(
        body,
        grid=(GRID,),
        in_specs=_p_specs() + [pl.BlockSpec((RB, 1), lambda i: (i, 0)),
                               _row_spec()],
        out_specs=[_row_spec(), _row_spec()],
        out_shape=[
            jax.ShapeDtypeStruct((N, FD), jnp.float32),
            jax.ShapeDtypeStruct((N, FD), jnp.float32),
        ],
    )(p, p, dis, tx0)


def _mm_stats(basis, W):
    K = len(basis)

    def body(*refs):
        tx_refs = refs[:K]
        w_ref = refs[K]
        h_ref, st_ref = refs[K + 1], refs[K + 2]
        acc_ref = refs[K + 3]
        i = pl.program_id(0)
        h = jnp.zeros((RB, FD), jnp.float32)
        for k in range(K):
            h = h + jnp.dot(tx_refs[k][...], w_ref[k],
                            preferred_element_type=jnp.float32)
        h_ref[...] = h

        @pl.when(i == 0)
        def _():
            acc_ref[...] = jnp.zeros((8, FD), jnp.float32)

        acc_ref[0:1, :] += jnp.sum(h, axis=0, keepdims=True)
        acc_ref[1:2, :] += jnp.sum(h * h, axis=0, keepdims=True)
        st_ref[...] = acc_ref[...]

    return pl.pallas_call(
        body,
        grid=(GRID,),
        in_specs=[_row_spec() for _ in range(K)]
        + [pl.BlockSpec((K, FD, FD), lambda i: (0, 0, 0))],
        out_specs=[_row_spec(), pl.BlockSpec((8, FD), lambda i: (0, 0))],
        out_shape=[
            jax.ShapeDtypeStruct((N, FD), jnp.float32),
            jax.ShapeDtypeStruct((8, FD), jnp.float32),
        ],
        scratch_shapes=[pltpu.VMEM((8, FD), jnp.float32)],
    )(*basis, W)


def _bn_relu_scale(h, st, dis):
    def body(h_ref, st_ref, dis_ref, out_ref, g_ref):
        mean = st_ref[0:1, :] * (1.0 / N)
        var = st_ref[1:2, :] * (1.0 / N) - mean * mean
        inv = lax.rsqrt(var + 1e-5)
        hr = jnp.maximum((h_ref[...] - mean) * inv, 0.0)
        out_ref[...] = hr
        g_ref[...] = dis_ref[...] * hr

    return pl.pallas_call(
        body,
        grid=(GRID,),
        in_specs=[_row_spec(), pl.BlockSpec((8, FD), lambda i: (0, 0)),
                  pl.BlockSpec((RB, 1), lambda i: (i, 0))],
        out_specs=[_row_spec(), _row_spec()],
        out_shape=[
            jax.ShapeDtypeStruct((N, FD), jnp.float32),
            jax.ShapeDtypeStruct((N, FD), jnp.float32),
        ],
    )(h, st, dis)


def _bn(h, st):
    def body(h_ref, st_ref, out_ref):
        mean = st_ref[0:1, :] * (1.0 / N)
        var = st_ref[1:2, :] * (1.0 / N) - mean * mean
        inv = lax.rsqrt(var + 1e-5)
        out_ref[...] = (h_ref[...] - mean) * inv

    return pl.pallas_call(
        body,
        grid=(GRID,),
        in_specs=[_row_spec(), pl.BlockSpec((8, FD), lambda i: (0, 0))],
        out_specs=_row_spec(),
        out_shape=jax.ShapeDtypeStruct((N, FD), jnp.float32),
    )(h, st)


def kernel(x, edge_index, W1, b1, Wmu, bmu, Wls, bls):
    src = edge_index[0]
    dst = edge_index[1]
    pad = EPAD - E
    pad_zero = jnp.zeros((pad,), jnp.int32)
    pad_dummy = jnp.full((pad,), N, jnp.int32)
    sidx = jnp.concatenate([src, pad_zero]).reshape(NW, TPW)
    didx = jnp.concatenate([dst, pad_dummy]).reshape(NW, TPW)
    sdeg = jnp.concatenate([src, pad_dummy]).reshape(NW, TPW)
    zeros128 = jnp.zeros((CHUNK, FD), jnp.float32)
    _spmm_sc = _sc_kernels()

    ones_tab = jnp.ones((N, FD), jnp.float32)
    degp = _spmm_sc(ones_tab, sidx, sdeg, zeros128)
    dis, g = _prep(degp, x)

    basis = [x]
    p = _spmm_sc(g, sidx, didx, zeros128)
    tx, g = _combine_first(p, dis)
    basis.append(tx)
    txm2, txm1 = basis[0], tx
    for _ in range(2, K1):
        p = _spmm_sc(g, sidx, didx, zeros128)
        tx, g = _combine_rec(p, dis, txm2)
        basis.append(tx)
        txm2, txm1 = txm1, tx
    h, st = _mm_stats(basis, W1)
    hr, g = _bn_relu_scale(h, st, dis)

    basis2 = [hr]
    p = _spmm_sc(g, sidx, didx, zeros128)
    tx, g = _combine_first(p, dis)
    basis2.append(tx)
    txm2, txm1 = basis2[0], tx
    for _ in range(2, K2):
        p = _spmm_sc(g, sidx, didx, zeros128)
        tx, g = _combine_rec(p, dis, txm2)
        basis2.append(tx)
        txm2, txm1 = txm1, tx
    Wc = jnp.concatenate([Wmu, Wls], axis=-1)
    h2, st2 = _mm_stats(basis2, Wc)
    out = _bn(h2, st2)
    return out[:, :OUT], out[:, OUT:]
